# baseline (speedup 1.0000x reference)
import functools

import jax
import jax.numpy as jnp
from jax import lax
from jax.experimental import pallas as pl
from jax.experimental.pallas import tpu as pltpu
from jax.experimental.pallas import tpu_sc as plsc

NUM_LAYER = 5
EMB = 128
D_EDGE = 16
N_NODES = 10000
N_EDGES = 320000
NUM_GRAPHS = 64

NC = 2
NS = 16
NW = NC * NS

CHUNK = 80
NCHUNK = 126
EW = CHUNK * NCHUNK
E_PAD = EW * NW
NSLOT = 3
NPAD = 10112
ROWS_PER_TILE = NPAD // NS
VEC = 16

_BPERM = tuple(
    32 * (j // 32) + (16 if j % 2 else 0) + (j % 32) // 2
    for j in range(EMB)
)


@functools.cache
def _make_sc_aggregate():
    mesh = plsc.VectorSubcoreMesh(core_axis_name="c", subcore_axis_name="s",
                                  num_cores=NC, num_subcores=NS)
    return pl.kernel(
        _sc_aggregate_body,
        out_type=jax.ShapeDtypeStruct((NC, NPAD, EMB), jnp.float32),
        mesh=mesh,
        scratch_types=[
            pltpu.VMEM((NSLOT, CHUNK), jnp.int32),
            pltpu.VMEM((NSLOT, CHUNK), jnp.int32),
            pltpu.VMEM((NSLOT, CHUNK, EMB), jnp.float32),
            pltpu.VMEM((CHUNK * EMB // 2,), jnp.int32),
            pltpu.VMEM((CHUNK * EMB // 2,), jnp.int32),
            pltpu.VMEM((CHUNK * EMB // 2,), jnp.int32),
            pltpu.VMEM_SHARED((NPAD, EMB), jnp.float32),
            pltpu.SemaphoreType.DMA((NSLOT,)),
            pltpu.SemaphoreType.DMA((NSLOT,)),
            pltpu.SemaphoreType.DMA((NSLOT,)),
            pltpu.SemaphoreType.DMA((NSLOT,)),
            pltpu.SemaphoreType.DMA((NSLOT,)),
        ],
    )


def _sc_aggregate_body(h_hbm, e_hbm, src_hbm, dst_hbm, out_hbm,
                       srcv, dstv, mv, ev0, ev1, ev2, acc,
                       sem_src, sem_dst, sem_e, sem_g, sem_sc):
    ev = (ev0, ev1, ev2)
    c = lax.axis_index("c")
    s = lax.axis_index("s")
    wid = s * NC + c
    edge_base = wid * EW
    row_base = s * ROWS_PER_TILE

    zero = jnp.zeros((VEC,), jnp.float32)

    def _zrow(r, carry):
        for b in range(NSLOT):
            for j in range(EMB // VEC):
                mv[b, r, pl.ds(j * VEC, VEC)] = zero
        return carry
    lax.fori_loop(0, CHUNK, _zrow, 0, unroll=2)

    for t in range(ROWS_PER_TILE // CHUNK):
        pltpu.sync_copy(mv.at[t % NSLOT],
                        acc.at[pl.ds(row_base + t * CHUNK, CHUNK)])
    rem = ROWS_PER_TILE % CHUNK
    pltpu.sync_copy(mv.at[0].at[pl.ds(0, rem)],
                    acc.at[pl.ds(row_base + (ROWS_PER_TILE // CHUNK) * CHUNK,
                                 rem)])
    plsc.subcore_barrier()

    def issue_inputs(k, b):
        base = edge_base + k * CHUNK
        pltpu.async_copy(src_hbm.at[pl.ds(base, CHUNK)], srcv.at[b],
                         sem_src.at[b])
        pltpu.async_copy(dst_hbm.at[pl.ds(base, CHUNK)], dstv.at[b],
                         sem_dst.at[b])
        pltpu.async_copy(e_hbm.at[pl.ds(base * (EMB // 2), CHUNK * EMB // 2)],
                         ev[b], sem_e.at[b])

    def wait_src(k, b):
        base = edge_base + k * CHUNK
        pltpu.make_async_copy(src_hbm.at[pl.ds(base, CHUNK)], srcv.at[b],
                              sem_src.at[b]).wait()

    def issue_gather(k, b):
        wait_src(k, b)
        pltpu.async_copy(h_hbm.at[srcv.at[b]], mv.at[b], sem_g.at[b])

    def wait_gather(b):
        pltpu.make_async_copy(h_hbm.at[srcv.at[b]], mv.at[b],
                              sem_g.at[b]).wait()

    def wait_e(k, b):
        base = edge_base + k * CHUNK
        pltpu.make_async_copy(
            e_hbm.at[pl.ds(base * (EMB // 2), CHUNK * EMB // 2)],
            ev[b], sem_e.at[b]).wait()

    def wait_dst(k, b):
        base = edge_base + k * CHUNK
        pltpu.make_async_copy(dst_hbm.at[pl.ds(base, CHUNK)], dstv.at[b],
                              sem_dst.at[b]).wait()

    def issue_scatter(b):
        pltpu.async_copy(mv.at[b], acc.at[dstv.at[b]], sem_sc.at[b],
                         add=True)

    def drain_scatter(b):
        pltpu.make_async_copy(mv.at[b], acc.at[dstv.at[b]],
                              sem_sc.at[b]).wait()

    def compute(k, b):
        wait_gather(b)
        wait_e(k, b)
        mask = jnp.int32(-65536)

        def _crow(r, inner):
            for t in range(EMB // (2 * VEC)):
                we = ev[b][pl.ds(r * (EMB // 2) + t * VEC, VEC)]
                ea = lax.bitcast_convert_type(we << 16, jnp.float32)
                eb = lax.bitcast_convert_type(we & mask, jnp.float32)
                sl0 = pl.ds(t * VEC, VEC)
                sl1 = pl.ds(EMB // 2 + t * VEC, VEC)
                mv[b, r, sl0] = jnp.maximum(mv[b, r, sl0] + ea, 0.0)
                mv[b, r, sl1] = jnp.maximum(mv[b, r, sl1] + eb, 0.0)
            return inner
        lax.fori_loop(0, CHUNK, _crow, 0, unroll=4)
        wait_dst(k, b)
        issue_scatter(b)

    issue_inputs(0, 0)
    issue_inputs(1, 1)
    issue_gather(0, 0)

    issue_gather(1, 1)
    compute(0, 0)
    issue_inputs(2, 2)

    def _main(i, carry):
        k0 = 1 + i * NSLOT
        for d in range(NSLOT):
            k = k0 + d
            b = (1 + d) % NSLOT
            bg = (2 + d) % NSLOT
            bi = d % NSLOT
            issue_gather(k + 1, bg)
            compute(k, b)
            drain_scatter(bi)
            issue_inputs(k + 2, bi)
        return carry
    lax.fori_loop(0, (NCHUNK - 3) // NSLOT, _main, 0)

    issue_gather(NCHUNK - 1, (NCHUNK - 1) % NSLOT)
    compute(NCHUNK - 2, (NCHUNK - 2) % NSLOT)
    drain_scatter((NCHUNK - 3) % NSLOT)
    compute(NCHUNK - 1, (NCHUNK - 1) % NSLOT)
    drain_scatter((NCHUNK - 2) % NSLOT)
    drain_scatter((NCHUNK - 1) % NSLOT)

    plsc.subcore_barrier()
    pltpu.sync_copy(acc.at[pl.ds(row_base, ROWS_PER_TILE)],
                    out_hbm.at[c, pl.ds(row_base, ROWS_PER_TILE)])


EBLK = 2048


def _encode_body(attr_ref, w_ref, b_ref, out_ref):
    e = jnp.dot(attr_ref[...], w_ref[...],
                preferred_element_type=jnp.float32) + b_ref[...]
    u = lax.bitcast_convert_type(e, jnp.int32)
    r = (u + jnp.int32(0x7FFF) + ((u >> 16) & 1)) >> 16
    lo = r[:, :EMB // 2] & jnp.int32(0xFFFF)
    hi = r[:, EMB // 2:] << 16
    out_ref[...] = lo | hi


def _encode(attr_pad, w, b):
    return pl.pallas_call(
        _encode_body,
        grid=(E_PAD // EBLK,),
        in_specs=[
            pl.BlockSpec((EBLK, D_EDGE), lambda i: (i, 0)),
            pl.BlockSpec((D_EDGE, EMB), lambda i: (0, 0)),
            pl.BlockSpec((1, EMB), lambda i: (0, 0)),
        ],
        out_specs=pl.BlockSpec((EBLK, EMB // 2), lambda i: (i, 0)),
        out_shape=jax.ShapeDtypeStruct((E_PAD, EMB // 2), jnp.int32),
    )(attr_pad, w, b.reshape(1, EMB))


NBLK = 400


def _mlp_body(h_ref, a_ref, scale_ref, w1_ref, b1_ref, g1_ref, be1_ref,
              w2_ref, b2_ref, gbn_ref, bbn_ref, out_ref, *, last):
    z = h_ref[...] * scale_ref[...] + a_ref[0] + a_ref[1]
    t = jnp.dot(z, w1_ref[...], preferred_element_type=jnp.float32)
    t = t + b1_ref[...]
    t = jnp.maximum(t * g1_ref[...] + be1_ref[...], 0.0)
    o = jnp.dot(t, w2_ref[...], preferred_element_type=jnp.float32)
    o = o + b2_ref[...]
    o = o * gbn_ref[...] + bbn_ref[...]
    if not last:
        o = jnp.maximum(o, 0.0)
    out_ref[...] = o


def _mlp(h, agg2, scale, w1, b1, g1, be1, w2, b2, gbn, bbn, last):
    body = functools.partial(_mlp_body, last=last)
    row = lambda v: v.reshape(1, -1)
    return pl.pallas_call(
        body,
        grid=(N_NODES // NBLK,),
        in_specs=[
            pl.BlockSpec((NBLK, EMB), lambda i: (i, 0)),
            pl.BlockSpec((NC, NBLK, EMB), lambda i: (0, i, 0)),
            pl.BlockSpec((1, EMB), lambda i: (0, 0)),
            pl.BlockSpec((EMB, 2 * EMB), lambda i: (0, 0)),
            pl.BlockSpec((1, 2 * EMB), lambda i: (0, 0)),
            pl.BlockSpec((1, 2 * EMB), lambda i: (0, 0)),
            pl.BlockSpec((1, 2 * EMB), lambda i: (0, 0)),
            pl.BlockSpec((2 * EMB, EMB), lambda i: (0, 0)),
            pl.BlockSpec((1, EMB), lambda i: (0, 0)),
            pl.BlockSpec((1, EMB), lambda i: (0, 0)),
            pl.BlockSpec((1, EMB), lambda i: (0, 0)),
        ],
        out_specs=pl.BlockSpec((NBLK, EMB), lambda i: (i, 0)),
        out_shape=jax.ShapeDtypeStruct((N_NODES, EMB), jnp.float32),
    )(h, agg2, scale, w1, row(b1), row(g1), row(be1), w2, row(b2),
      row(gbn), row(bbn))


PBLK = 2000


def _pool_body(h_ref, batch_ref, out_ref, sums_ref, cnts_ref):
    i = pl.program_id(0)
    gids = lax.broadcasted_iota(jnp.int32, (NUM_GRAPHS, PBLK), 0)
    oh = (gids == batch_ref[0]).astype(jnp.float32)
    psum = jnp.dot(oh, h_ref[...], preferred_element_type=jnp.float32)
    pcnt = jnp.broadcast_to(jnp.sum(oh, axis=1, keepdims=True),
                            (NUM_GRAPHS, EMB))

    @pl.when(i == 0)
    def _init():
        sums_ref[...] = psum
        cnts_ref[...] = pcnt

    @pl.when(i > 0)
    def _accum():
        sums_ref[...] += psum
        cnts_ref[...] += pcnt

    @pl.when(i == pl.num_programs(0) - 1)
    def _final():
        out_ref[...] = sums_ref[...] / jnp.maximum(cnts_ref[...], 1.0)


def _pool(h, batch2d):
    return pl.pallas_call(
        _pool_body,
        grid=(N_NODES // PBLK,),
        in_specs=[
            pl.BlockSpec((PBLK, EMB), lambda i: (i, 0)),
            pl.BlockSpec((1, 1, PBLK), lambda i: (i, 0, 0)),
        ],
        out_specs=pl.BlockSpec((NUM_GRAPHS, EMB), lambda i: (0, 0)),
        out_shape=jax.ShapeDtypeStruct((NUM_GRAPHS, EMB), jnp.float32),
        scratch_shapes=[
            pltpu.VMEM((NUM_GRAPHS, EMB), jnp.float32),
            pltpu.VMEM((NUM_GRAPHS, EMB), jnp.float32),
        ],
    )(h, batch2d)


def kernel(x, edge_attr, W_edge, b_edge, eps, W1, b1, g1, be1, W2, b2,
           g_bn, b_bn, edge_index, batch):
    src = edge_index[0].astype(jnp.int32)
    dst = edge_index[1].astype(jnp.int32)
    pad = E_PAD - N_EDGES
    src = jnp.pad(src, (0, pad))
    dst = jnp.pad(dst, (0, pad), constant_values=N_NODES)
    attr_pad = jnp.pad(edge_attr, ((0, pad), (0, 0)))
    batch2d = batch.astype(jnp.int32).reshape(N_NODES // PBLK, 1, PBLK)

    es = [_encode(attr_pad, W_edge[l], b_edge[l]).reshape(E_PAD * EMB // 2)
          for l in range(NUM_LAYER)]
    h = x
    for l in range(NUM_LAYER):
        agg2 = _make_sc_aggregate()(h, es[l], src, dst)
        scale = jnp.full((1, EMB), 1.0 + eps[l], dtype=jnp.float32)
        h = _mlp(h, agg2, scale, W1[l], b1[l], g1[l], be1[l],
                 W2[l], b2[l], g_bn[l], b_bn[l], last=(l == NUM_LAYER - 1))

    return _pool(h, batch2d)


# speedup vs baseline: 1.0002x; 1.0002x over previous
import functools

import jax
import jax.numpy as jnp
from jax import lax
from jax.experimental import pallas as pl
from jax.experimental.pallas import tpu as pltpu
from jax.experimental.pallas import tpu_sc as plsc

NUM_LAYER = 5
EMB = 128
D_EDGE = 16
N_NODES = 10000
N_EDGES = 320000
NUM_GRAPHS = 64

NC = 2
NS = 16
NW = NC * NS

CHUNK = 80
NCHUNK = 126
EW = CHUNK * NCHUNK
E_PAD = EW * NW
NSLOT = 3
NPAD = 10112
ROWS_PER_TILE = NPAD // NS
VEC = 16

_BPERM = tuple(
    32 * (j // 32) + (16 if j % 2 else 0) + (j % 32) // 2
    for j in range(EMB)
)


@functools.cache
def _make_sc_aggregate():
    mesh = plsc.VectorSubcoreMesh(core_axis_name="c", subcore_axis_name="s",
                                  num_cores=NC, num_subcores=NS)
    return pl.kernel(
        _sc_aggregate_body,
        out_type=jax.ShapeDtypeStruct((NC, NPAD, EMB), jnp.float32),
        mesh=mesh,
        scratch_types=[
            pltpu.VMEM((NSLOT, CHUNK), jnp.int32),
            pltpu.VMEM((NSLOT, CHUNK), jnp.int32),
            pltpu.VMEM((NSLOT, CHUNK, EMB), jnp.float32),
            pltpu.VMEM((CHUNK * EMB // 2,), jnp.int32),
            pltpu.VMEM((CHUNK * EMB // 2,), jnp.int32),
            pltpu.VMEM((CHUNK * EMB // 2,), jnp.int32),
            pltpu.VMEM_SHARED((NPAD, EMB), jnp.float32),
            pltpu.SemaphoreType.DMA((NSLOT,)),
            pltpu.SemaphoreType.DMA((NSLOT,)),
            pltpu.SemaphoreType.DMA((NSLOT,)),
            pltpu.SemaphoreType.DMA((NSLOT,)),
            pltpu.SemaphoreType.DMA((NSLOT,)),
        ],
    )


def _sc_aggregate_body(h_hbm, e_hbm, src_hbm, dst_hbm, out_hbm,
                       srcv, dstv, mv, ev0, ev1, ev2, acc,
                       sem_src, sem_dst, sem_e, sem_g, sem_sc):
    ev = (ev0, ev1, ev2)
    c = lax.axis_index("c")
    s = lax.axis_index("s")
    wid = s * NC + c
    edge_base = wid * EW
    row_base = s * ROWS_PER_TILE

    zero = jnp.zeros((VEC,), jnp.float32)

    def _zrow(r, carry):
        for b in range(NSLOT):
            for j in range(EMB // VEC):
                mv[b, r, pl.ds(j * VEC, VEC)] = zero
        return carry
    lax.fori_loop(0, CHUNK, _zrow, 0, unroll=2)

    for t in range(ROWS_PER_TILE // CHUNK):
        pltpu.sync_copy(mv.at[t % NSLOT],
                        acc.at[pl.ds(row_base + t * CHUNK, CHUNK)])
    rem = ROWS_PER_TILE % CHUNK
    pltpu.sync_copy(mv.at[0].at[pl.ds(0, rem)],
                    acc.at[pl.ds(row_base + (ROWS_PER_TILE // CHUNK) * CHUNK,
                                 rem)])
    plsc.subcore_barrier()

    def issue_inputs(k, b):
        base = edge_base + k * CHUNK
        pltpu.async_copy(src_hbm.at[pl.ds(base, CHUNK)], srcv.at[b],
                         sem_src.at[b])
        pltpu.async_copy(dst_hbm.at[pl.ds(base, CHUNK)], dstv.at[b],
                         sem_dst.at[b])
        pltpu.async_copy(e_hbm.at[pl.ds(base * (EMB // 2), CHUNK * EMB // 2)],
                         ev[b], sem_e.at[b])

    def wait_src(k, b):
        base = edge_base + k * CHUNK
        pltpu.make_async_copy(src_hbm.at[pl.ds(base, CHUNK)], srcv.at[b],
                              sem_src.at[b]).wait()

    HC = CHUNK // 2

    def issue_gather(k, b):
        wait_src(k, b)
        pltpu.async_copy(h_hbm.at[srcv.at[b].at[pl.ds(0, HC)]],
                         mv.at[b].at[pl.ds(0, HC)], sem_g.at[b])
        pltpu.async_copy(h_hbm.at[srcv.at[b].at[pl.ds(HC, HC)]],
                         mv.at[b].at[pl.ds(HC, HC)], sem_g.at[b])

    def wait_gather(b):
        for o in (0, HC):
            pltpu.make_async_copy(h_hbm.at[srcv.at[b].at[pl.ds(o, HC)]],
                                  mv.at[b].at[pl.ds(o, HC)],
                                  sem_g.at[b]).wait()

    def wait_e(k, b):
        base = edge_base + k * CHUNK
        pltpu.make_async_copy(
            e_hbm.at[pl.ds(base * (EMB // 2), CHUNK * EMB // 2)],
            ev[b], sem_e.at[b]).wait()

    def wait_dst(k, b):
        base = edge_base + k * CHUNK
        pltpu.make_async_copy(dst_hbm.at[pl.ds(base, CHUNK)], dstv.at[b],
                              sem_dst.at[b]).wait()

    def issue_scatter(b):
        pltpu.async_copy(mv.at[b], acc.at[dstv.at[b]], sem_sc.at[b],
                         add=True)

    def drain_scatter(b):
        pltpu.make_async_copy(mv.at[b], acc.at[dstv.at[b]],
                              sem_sc.at[b]).wait()

    def compute(k, b):
        wait_gather(b)
        wait_e(k, b)
        mask = jnp.int32(-65536)

        def _crow(r, inner):
            for t in range(EMB // (2 * VEC)):
                we = ev[b][pl.ds(r * (EMB // 2) + t * VEC, VEC)]
                ea = lax.bitcast_convert_type(we << 16, jnp.float32)
                eb = lax.bitcast_convert_type(we & mask, jnp.float32)
                sl0 = pl.ds(t * VEC, VEC)
                sl1 = pl.ds(EMB // 2 + t * VEC, VEC)
                mv[b, r, sl0] = jnp.maximum(mv[b, r, sl0] + ea, 0.0)
                mv[b, r, sl1] = jnp.maximum(mv[b, r, sl1] + eb, 0.0)
            return inner
        lax.fori_loop(0, CHUNK, _crow, 0, unroll=4)
        wait_dst(k, b)
        issue_scatter(b)

    issue_inputs(0, 0)
    issue_inputs(1, 1)
    issue_gather(0, 0)

    issue_gather(1, 1)
    compute(0, 0)
    issue_inputs(2, 2)

    def _main(i, carry):
        k0 = 1 + i * NSLOT
        for d in range(NSLOT):
            k = k0 + d
            b = (1 + d) % NSLOT
            bg = (2 + d) % NSLOT
            bi = d % NSLOT
            issue_gather(k + 1, bg)
            compute(k, b)
            drain_scatter(bi)
            issue_inputs(k + 2, bi)
        return carry
    lax.fori_loop(0, (NCHUNK - 3) // NSLOT, _main, 0)

    issue_gather(NCHUNK - 1, (NCHUNK - 1) % NSLOT)
    compute(NCHUNK - 2, (NCHUNK - 2) % NSLOT)
    drain_scatter((NCHUNK - 3) % NSLOT)
    compute(NCHUNK - 1, (NCHUNK - 1) % NSLOT)
    drain_scatter((NCHUNK - 2) % NSLOT)
    drain_scatter((NCHUNK - 1) % NSLOT)

    plsc.subcore_barrier()
    pltpu.sync_copy(acc.at[pl.ds(row_base, ROWS_PER_TILE)],
                    out_hbm.at[c, pl.ds(row_base, ROWS_PER_TILE)])


EBLK = 2048


def _encode_body(attr_ref, w_ref, b_ref, out_ref):
    e = jnp.dot(attr_ref[...], w_ref[...],
                preferred_element_type=jnp.float32) + b_ref[...]
    u = lax.bitcast_convert_type(e, jnp.int32)
    r = (u + jnp.int32(0x7FFF) + ((u >> 16) & 1)) >> 16
    lo = r[:, :EMB // 2] & jnp.int32(0xFFFF)
    hi = r[:, EMB // 2:] << 16
    out_ref[...] = lo | hi


def _encode(attr_pad, w, b):
    return pl.pallas_call(
        _encode_body,
        grid=(E_PAD // EBLK,),
        in_specs=[
            pl.BlockSpec((EBLK, D_EDGE), lambda i: (i, 0)),
            pl.BlockSpec((D_EDGE, EMB), lambda i: (0, 0)),
            pl.BlockSpec((1, EMB), lambda i: (0, 0)),
        ],
        out_specs=pl.BlockSpec((EBLK, EMB // 2), lambda i: (i, 0)),
        out_shape=jax.ShapeDtypeStruct((E_PAD, EMB // 2), jnp.int32),
    )(attr_pad, w, b.reshape(1, EMB))


NBLK = 400


def _mlp_body(h_ref, a_ref, scale_ref, w1_ref, b1_ref, g1_ref, be1_ref,
              w2_ref, b2_ref, gbn_ref, bbn_ref, out_ref, *, last):
    z = h_ref[...] * scale_ref[...] + a_ref[0] + a_ref[1]
    t = jnp.dot(z, w1_ref[...], preferred_element_type=jnp.float32)
    t = t + b1_ref[...]
    t = jnp.maximum(t * g1_ref[...] + be1_ref[...], 0.0)
    o = jnp.dot(t, w2_ref[...], preferred_element_type=jnp.float32)
    o = o + b2_ref[...]
    o = o * gbn_ref[...] + bbn_ref[...]
    if not last:
        o = jnp.maximum(o, 0.0)
    out_ref[...] = o


def _mlp(h, agg2, scale, w1, b1, g1, be1, w2, b2, gbn, bbn, last):
    body = functools.partial(_mlp_body, last=last)
    row = lambda v: v.reshape(1, -1)
    return pl.pallas_call(
        body,
        grid=(N_NODES // NBLK,),
        in_specs=[
            pl.BlockSpec((NBLK, EMB), lambda i: (i, 0)),
            pl.BlockSpec((NC, NBLK, EMB), lambda i: (0, i, 0)),
            pl.BlockSpec((1, EMB), lambda i: (0, 0)),
            pl.BlockSpec((EMB, 2 * EMB), lambda i: (0, 0)),
            pl.BlockSpec((1, 2 * EMB), lambda i: (0, 0)),
            pl.BlockSpec((1, 2 * EMB), lambda i: (0, 0)),
            pl.BlockSpec((1, 2 * EMB), lambda i: (0, 0)),
            pl.BlockSpec((2 * EMB, EMB), lambda i: (0, 0)),
            pl.BlockSpec((1, EMB), lambda i: (0, 0)),
            pl.BlockSpec((1, EMB), lambda i: (0, 0)),
            pl.BlockSpec((1, EMB), lambda i: (0, 0)),
        ],
        out_specs=pl.BlockSpec((NBLK, EMB), lambda i: (i, 0)),
        out_shape=jax.ShapeDtypeStruct((N_NODES, EMB), jnp.float32),
    )(h, agg2, scale, w1, row(b1), row(g1), row(be1), w2, row(b2),
      row(gbn), row(bbn))


PBLK = 2000


def _pool_body(h_ref, batch_ref, out_ref, sums_ref, cnts_ref):
    i = pl.program_id(0)
    gids = lax.broadcasted_iota(jnp.int32, (NUM_GRAPHS, PBLK), 0)
    oh = (gids == batch_ref[0]).astype(jnp.float32)
    psum = jnp.dot(oh, h_ref[...], preferred_element_type=jnp.float32)
    pcnt = jnp.broadcast_to(jnp.sum(oh, axis=1, keepdims=True),
                            (NUM_GRAPHS, EMB))

    @pl.when(i == 0)
    def _init():
        sums_ref[...] = psum
        cnts_ref[...] = pcnt

    @pl.when(i > 0)
    def _accum():
        sums_ref[...] += psum
        cnts_ref[...] += pcnt

    @pl.when(i == pl.num_programs(0) - 1)
    def _final():
        out_ref[...] = sums_ref[...] / jnp.maximum(cnts_ref[...], 1.0)


def _pool(h, batch2d):
    return pl.pallas_call(
        _pool_body,
        grid=(N_NODES // PBLK,),
        in_specs=[
            pl.BlockSpec((PBLK, EMB), lambda i: (i, 0)),
            pl.BlockSpec((1, 1, PBLK), lambda i: (i, 0, 0)),
        ],
        out_specs=pl.BlockSpec((NUM_GRAPHS, EMB), lambda i: (0, 0)),
        out_shape=jax.ShapeDtypeStruct((NUM_GRAPHS, EMB), jnp.float32),
        scratch_shapes=[
            pltpu.VMEM((NUM_GRAPHS, EMB), jnp.float32),
            pltpu.VMEM((NUM_GRAPHS, EMB), jnp.float32),
        ],
    )(h, batch2d)


def kernel(x, edge_attr, W_edge, b_edge, eps, W1, b1, g1, be1, W2, b2,
           g_bn, b_bn, edge_index, batch):
    src = edge_index[0].astype(jnp.int32)
    dst = edge_index[1].astype(jnp.int32)
    pad = E_PAD - N_EDGES
    src = jnp.pad(src, (0, pad))
    dst = jnp.pad(dst, (0, pad), constant_values=N_NODES)
    attr_pad = jnp.pad(edge_attr, ((0, pad), (0, 0)))
    batch2d = batch.astype(jnp.int32).reshape(N_NODES // PBLK, 1, PBLK)

    h = x
    for l in range(NUM_LAYER):
        e = _encode(attr_pad, W_edge[l],
                    b_edge[l]).reshape(E_PAD * EMB // 2)
        agg2 = _make_sc_aggregate()(h, e, src, dst)
        scale = jnp.full((1, EMB), 1.0 + eps[l], dtype=jnp.float32)
        h = _mlp(h, agg2, scale, W1[l], b1[l], g1[l], be1[l],
                 W2[l], b2[l], g_bn[l], b_bn[l], last=(l == NUM_LAYER - 1))

    return _pool(h, batch2d)


# speedup vs baseline: 1.0354x; 1.0352x over previous
import functools

import jax
import jax.numpy as jnp
from jax import lax
from jax.experimental import pallas as pl
from jax.experimental.pallas import tpu as pltpu
from jax.experimental.pallas import tpu_sc as plsc

NUM_LAYER = 5
EMB = 128
D_EDGE = 16
N_NODES = 10000
N_EDGES = 320000
NUM_GRAPHS = 64

NC = 2
NS = 16
NW = NC * NS

CHUNK = 80
NCHUNK0 = 141
NCHUNK1 = 111
EW0 = CHUNK * NCHUNK0
EW1 = CHUNK * NCHUNK1
E_PAD = (EW0 + EW1) * NS
NSLOT = 3
NPAD = 10112
ROWS_PER_TILE = NPAD // NS
VEC = 16

_BPERM = tuple(
    32 * (j // 32) + (16 if j % 2 else 0) + (j % 32) // 2
    for j in range(EMB)
)


@functools.cache
def _make_sc_aggregate():
    mesh = plsc.VectorSubcoreMesh(core_axis_name="c", subcore_axis_name="s",
                                  num_cores=NC, num_subcores=NS)
    return pl.kernel(
        _sc_aggregate_body,
        out_type=jax.ShapeDtypeStruct((NC, NPAD, EMB), jnp.float32),
        mesh=mesh,
        scratch_types=[
            pltpu.VMEM((NSLOT, CHUNK), jnp.int32),
            pltpu.VMEM((NSLOT, CHUNK), jnp.int32),
            pltpu.VMEM((NSLOT, CHUNK, EMB), jnp.float32),
            pltpu.VMEM((CHUNK * EMB // 2,), jnp.int32),
            pltpu.VMEM((CHUNK * EMB // 2,), jnp.int32),
            pltpu.VMEM((CHUNK * EMB // 2,), jnp.int32),
            pltpu.VMEM_SHARED((NPAD, EMB), jnp.float32),
            pltpu.SemaphoreType.DMA((NSLOT,)),
            pltpu.SemaphoreType.DMA((NSLOT,)),
            pltpu.SemaphoreType.DMA((NSLOT,)),
            pltpu.SemaphoreType.DMA((NSLOT,)),
            pltpu.SemaphoreType.DMA((NSLOT,)),
        ],
    )


def _sc_aggregate_body(h_hbm, e_hbm, src_hbm, dst_hbm, out_hbm,
                       srcv, dstv, mv, ev0, ev1, ev2, acc,
                       sem_src, sem_dst, sem_e, sem_g, sem_sc):
    ev = (ev0, ev1, ev2)
    c = lax.axis_index("c")
    s = lax.axis_index("s")
    edge_base = jnp.where(c == 0, s * EW0, NS * EW0 + s * EW1)
    nchunk = jnp.where(c == 0, NCHUNK0, NCHUNK1)
    row_base = s * ROWS_PER_TILE

    zero = jnp.zeros((VEC,), jnp.float32)

    def _zrow(r, carry):
        for b in range(NSLOT):
            for j in range(EMB // VEC):
                mv[b, r, pl.ds(j * VEC, VEC)] = zero
        return carry
    lax.fori_loop(0, CHUNK, _zrow, 0, unroll=2)

    for t in range(ROWS_PER_TILE // CHUNK):
        pltpu.sync_copy(mv.at[t % NSLOT],
                        acc.at[pl.ds(row_base + t * CHUNK, CHUNK)])
    rem = ROWS_PER_TILE % CHUNK
    pltpu.sync_copy(mv.at[0].at[pl.ds(0, rem)],
                    acc.at[pl.ds(row_base + (ROWS_PER_TILE // CHUNK) * CHUNK,
                                 rem)])
    plsc.subcore_barrier()

    def issue_inputs(k, b):
        base = edge_base + k * CHUNK
        pltpu.async_copy(src_hbm.at[pl.ds(base, CHUNK)], srcv.at[b],
                         sem_src.at[b])
        pltpu.async_copy(dst_hbm.at[pl.ds(base, CHUNK)], dstv.at[b],
                         sem_dst.at[b])
        pltpu.async_copy(e_hbm.at[pl.ds(base * (EMB // 2), CHUNK * EMB // 2)],
                         ev[b], sem_e.at[b])

    def wait_src(k, b):
        base = edge_base + k * CHUNK
        pltpu.make_async_copy(src_hbm.at[pl.ds(base, CHUNK)], srcv.at[b],
                              sem_src.at[b]).wait()

    def issue_gather(k, b):
        wait_src(k, b)
        pltpu.async_copy(h_hbm.at[srcv.at[b]], mv.at[b], sem_g.at[b])

    def wait_gather(b):
        pltpu.make_async_copy(h_hbm.at[srcv.at[b]], mv.at[b],
                              sem_g.at[b]).wait()

    def wait_e(k, b):
        base = edge_base + k * CHUNK
        pltpu.make_async_copy(
            e_hbm.at[pl.ds(base * (EMB // 2), CHUNK * EMB // 2)],
            ev[b], sem_e.at[b]).wait()

    def wait_dst(k, b):
        base = edge_base + k * CHUNK
        pltpu.make_async_copy(dst_hbm.at[pl.ds(base, CHUNK)], dstv.at[b],
                              sem_dst.at[b]).wait()

    def issue_scatter(b):
        pltpu.async_copy(mv.at[b], acc.at[dstv.at[b]], sem_sc.at[b],
                         add=True)

    def drain_scatter(b):
        pltpu.make_async_copy(mv.at[b], acc.at[dstv.at[b]],
                              sem_sc.at[b]).wait()

    def compute(k, b):
        wait_gather(b)
        wait_e(k, b)
        mask = jnp.int32(-65536)

        def _crow(r, inner):
            for t in range(EMB // (2 * VEC)):
                we = ev[b][pl.ds(r * (EMB // 2) + t * VEC, VEC)]
                ea = lax.bitcast_convert_type(we << 16, jnp.float32)
                eb = lax.bitcast_convert_type(we & mask, jnp.float32)
                sl0 = pl.ds(t * VEC, VEC)
                sl1 = pl.ds(EMB // 2 + t * VEC, VEC)
                mv[b, r, sl0] = jnp.maximum(mv[b, r, sl0] + ea, 0.0)
                mv[b, r, sl1] = jnp.maximum(mv[b, r, sl1] + eb, 0.0)
            return inner
        lax.fori_loop(0, CHUNK, _crow, 0, unroll=4)
        wait_dst(k, b)
        issue_scatter(b)

    issue_inputs(0, 0)
    issue_inputs(1, 1)
    issue_gather(0, 0)

    issue_gather(1, 1)
    compute(0, 0)
    issue_inputs(2, 2)

    def _main(i, carry):
        k0 = 1 + i * NSLOT
        for d in range(NSLOT):
            k = k0 + d
            b = (1 + d) % NSLOT
            bg = (2 + d) % NSLOT
            bi = d % NSLOT
            issue_gather(k + 1, bg)
            compute(k, b)
            drain_scatter(bi)
            issue_inputs(k + 2, bi)
        return carry
    main_t = jnp.where(c == 0, (NCHUNK0 - 3) // NSLOT,
                       (NCHUNK1 - 3) // NSLOT)
    lax.fori_loop(0, main_t, _main, 0)

    issue_gather(nchunk - 1, 2)
    compute(nchunk - 2, 1)
    drain_scatter(0)
    compute(nchunk - 1, 2)
    drain_scatter(1)
    drain_scatter(2)

    plsc.subcore_barrier()
    pltpu.sync_copy(acc.at[pl.ds(row_base, ROWS_PER_TILE)],
                    out_hbm.at[c, pl.ds(row_base, ROWS_PER_TILE)])


EBLK = 2048


def _encode_body(attr_ref, w_ref, b_ref, out_ref):
    e = jnp.dot(attr_ref[...], w_ref[...],
                preferred_element_type=jnp.float32) + b_ref[...]
    u = lax.bitcast_convert_type(e, jnp.int32)
    r = (u + jnp.int32(0x7FFF) + ((u >> 16) & 1)) >> 16
    lo = r[:, :EMB // 2] & jnp.int32(0xFFFF)
    hi = r[:, EMB // 2:] << 16
    out_ref[...] = lo | hi


def _encode(attr_pad, w, b):
    return pl.pallas_call(
        _encode_body,
        grid=(E_PAD // EBLK,),
        in_specs=[
            pl.BlockSpec((EBLK, D_EDGE), lambda i: (i, 0)),
            pl.BlockSpec((D_EDGE, EMB), lambda i: (0, 0)),
            pl.BlockSpec((1, EMB), lambda i: (0, 0)),
        ],
        out_specs=pl.BlockSpec((EBLK, EMB // 2), lambda i: (i, 0)),
        out_shape=jax.ShapeDtypeStruct((E_PAD, EMB // 2), jnp.int32),
    )(attr_pad, w, b.reshape(1, EMB))


NBLK = 400


def _mlp_body(h_ref, a_ref, scale_ref, w1_ref, b1_ref, g1_ref, be1_ref,
              w2_ref, b2_ref, gbn_ref, bbn_ref, out_ref, *, last):
    z = h_ref[...] * scale_ref[...] + a_ref[0] + a_ref[1]
    t = jnp.dot(z, w1_ref[...], preferred_element_type=jnp.float32)
    t = t + b1_ref[...]
    t = jnp.maximum(t * g1_ref[...] + be1_ref[...], 0.0)
    o = jnp.dot(t, w2_ref[...], preferred_element_type=jnp.float32)
    o = o + b2_ref[...]
    o = o * gbn_ref[...] + bbn_ref[...]
    if not last:
        o = jnp.maximum(o, 0.0)
    out_ref[...] = o


def _mlp(h, agg2, scale, w1, b1, g1, be1, w2, b2, gbn, bbn, last):
    body = functools.partial(_mlp_body, last=last)
    row = lambda v: v.reshape(1, -1)
    return pl.pallas_call(
        body,
        grid=(N_NODES // NBLK,),
        in_specs=[
            pl.BlockSpec((NBLK, EMB), lambda i: (i, 0)),
            pl.BlockSpec((NC, NBLK, EMB), lambda i: (0, i, 0)),
            pl.BlockSpec((1, EMB), lambda i: (0, 0)),
            pl.BlockSpec((EMB, 2 * EMB), lambda i: (0, 0)),
            pl.BlockSpec((1, 2 * EMB), lambda i: (0, 0)),
            pl.BlockSpec((1, 2 * EMB), lambda i: (0, 0)),
            pl.BlockSpec((1, 2 * EMB), lambda i: (0, 0)),
            pl.BlockSpec((2 * EMB, EMB), lambda i: (0, 0)),
            pl.BlockSpec((1, EMB), lambda i: (0, 0)),
            pl.BlockSpec((1, EMB), lambda i: (0, 0)),
            pl.BlockSpec((1, EMB), lambda i: (0, 0)),
        ],
        out_specs=pl.BlockSpec((NBLK, EMB), lambda i: (i, 0)),
        out_shape=jax.ShapeDtypeStruct((N_NODES, EMB), jnp.float32),
    )(h, agg2, scale, w1, row(b1), row(g1), row(be1), w2, row(b2),
      row(gbn), row(bbn))


PBLK = 2000


def _pool_body(h_ref, batch_ref, out_ref, sums_ref, cnts_ref):
    i = pl.program_id(0)
    gids = lax.broadcasted_iota(jnp.int32, (NUM_GRAPHS, PBLK), 0)
    oh = (gids == batch_ref[0]).astype(jnp.float32)
    psum = jnp.dot(oh, h_ref[...], preferred_element_type=jnp.float32)
    pcnt = jnp.broadcast_to(jnp.sum(oh, axis=1, keepdims=True),
                            (NUM_GRAPHS, EMB))

    @pl.when(i == 0)
    def _init():
        sums_ref[...] = psum
        cnts_ref[...] = pcnt

    @pl.when(i > 0)
    def _accum():
        sums_ref[...] += psum
        cnts_ref[...] += pcnt

    @pl.when(i == pl.num_programs(0) - 1)
    def _final():
        out_ref[...] = sums_ref[...] / jnp.maximum(cnts_ref[...], 1.0)


def _pool(h, batch2d):
    return pl.pallas_call(
        _pool_body,
        grid=(N_NODES // PBLK,),
        in_specs=[
            pl.BlockSpec((PBLK, EMB), lambda i: (i, 0)),
            pl.BlockSpec((1, 1, PBLK), lambda i: (i, 0, 0)),
        ],
        out_specs=pl.BlockSpec((NUM_GRAPHS, EMB), lambda i: (0, 0)),
        out_shape=jax.ShapeDtypeStruct((NUM_GRAPHS, EMB), jnp.float32),
        scratch_shapes=[
            pltpu.VMEM((NUM_GRAPHS, EMB), jnp.float32),
            pltpu.VMEM((NUM_GRAPHS, EMB), jnp.float32),
        ],
    )(h, batch2d)


def kernel(x, edge_attr, W_edge, b_edge, eps, W1, b1, g1, be1, W2, b2,
           g_bn, b_bn, edge_index, batch):
    src = edge_index[0].astype(jnp.int32)
    dst = edge_index[1].astype(jnp.int32)
    pad = E_PAD - N_EDGES
    src = jnp.pad(src, (0, pad))
    dst = jnp.pad(dst, (0, pad), constant_values=N_NODES)
    attr_pad = jnp.pad(edge_attr, ((0, pad), (0, 0)))
    batch2d = batch.astype(jnp.int32).reshape(N_NODES // PBLK, 1, PBLK)

    h = x
    for l in range(NUM_LAYER):
        e = _encode(attr_pad, W_edge[l],
                    b_edge[l]).reshape(E_PAD * EMB // 2)
        agg2 = _make_sc_aggregate()(h, e, src, dst)
        scale = jnp.full((1, EMB), 1.0 + eps[l], dtype=jnp.float32)
        h = _mlp(h, agg2, scale, W1[l], b1[l], g1[l], be1[l],
                 W2[l], b2[l], g_bn[l], b_bn[l], last=(l == NUM_LAYER - 1))

    return _pool(h, batch2d)
